# baseline (speedup 1.0000x reference)
import functools

import jax
import jax.numpy as jnp
import numpy as np
from jax import lax
from jax.experimental import pallas as pl
from jax.experimental.pallas import tpu as pltpu
from jax.experimental.pallas import tpu_sc as plsc

N_VARS = 25000
N_CLAUSES = 50000
N_LITS = 50000
N_EDGES = 800000
FEATURE_MAPS = 64
QUERY_MAPS = 64
ROUNDS = 4

_F32 = jnp.float32
_I32 = jnp.int32


_NC, _NS = 2, 16
_EPAD = 802816
_NROWS = 50000
_NDUMP = 50048
_ZSTRIPE = _NDUMP // _NS
_DUMP = _NROWS


@functools.lru_cache(maxsize=None)
def _make_seg_kernel(fh, split_features):
    chunk = 392 if split_features else 896
    per_tile = _EPAD // _NS if split_features else _EPAD // (_NC * _NS)
    nchunks = per_tile // chunk
    mesh = plsc.VectorSubcoreMesh(
        core_axis_name="c", subcore_axis_name="s",
        num_cores=_NC, num_subcores=_NS)

    @functools.partial(
        pl.kernel,
        out_type=jax.ShapeDtypeStruct((_NC, _NDUMP, fh), _F32),
        mesh=mesh,
        scratch_types=(
            [pltpu.VMEM_SHARED((_NDUMP, fh), _F32)]
            + [pltpu.VMEM((chunk, fh), _F32)] * 2
            + [pltpu.VMEM((chunk,), _I32)] * 8
            + [pltpu.SemaphoreType.DMA] * 3
        ),
        compiler_params=pltpu.CompilerParams(use_tc_tiling_on_sc=False),
    )
    def seg(tabA, tabB, gidx, sidx, zeros, out, acc, gbuf0, gbuf1,
            giv0, giv1, giv2, giv3, siv0, siv1, siv2, siv3,
            gsem, ssem, isem):
        c = lax.axis_index("c")
        s = lax.axis_index("s")
        pltpu.sync_copy(zeros, acc.at[pl.ds(s * _ZSTRIPE, _ZSTRIPE)])
        plsc.subcore_barrier()
        if split_features:
            base = s * per_tile
        else:
            base = (c * _NS + s) * per_tile
        gbufs = (gbuf0, gbuf1)
        givs = (giv0, giv1, giv2, giv3)
        sivs = (siv0, siv1, siv2, siv3)
        dummy = tabA.at[pl.ds(0, chunk)]
        idummy = gidx.at[pl.ds(0, chunk)]

        def idx_start(i, r):
            off = base + i * chunk
            pltpu.async_copy(gidx.at[pl.ds(off, chunk)], givs[r], isem)
            pltpu.async_copy(sidx.at[pl.ds(off, chunk)], sivs[r], isem)

        def iwait(r):
            pltpu.make_async_copy(idummy, givs[r], isem).wait()
            pltpu.make_async_copy(idummy, sivs[r], isem).wait()

        def gather_start(r, b):
            if split_features:
                @pl.when(c == 0)
                def _():
                    pltpu.async_copy(tabA.at[givs[r]], gbufs[b], gsem)

                @pl.when(c == 1)
                def _():
                    pltpu.async_copy(tabB.at[givs[r]], gbufs[b], gsem)
            else:
                pltpu.async_copy(tabA.at[givs[r]], gbufs[b], gsem)

        def gwait(b):
            pltpu.make_async_copy(dummy, gbufs[b], gsem).wait()

        def swait(b):
            pltpu.make_async_copy(dummy, gbufs[b], ssem).wait()

        idx_start(0, 0)
        idx_start(1, 1)
        iwait(0)
        gather_start(0, 0)

        def group(g, carry):
            for sub in range(4):
                i = 4 * g + sub
                b = sub % 2
                gwait(b)
                pltpu.async_copy(gbufs[b], acc.at[sivs[sub]], ssem, add=True)

                @pl.when(i >= 1)
                def _():
                    swait(1 - b)

                @pl.when(i + 2 < nchunks)
                def _():
                    idx_start(i + 2, (sub + 2) % 4)

                @pl.when(i + 1 < nchunks)
                def _():
                    iwait((sub + 1) % 4)
                    gather_start((sub + 1) % 4, 1 - b)
            return carry

        lax.fori_loop(0, nchunks // 4, group, 0)
        swait((nchunks - 1) % 2)
        plsc.subcore_barrier()
        pltpu.sync_copy(acc.at[pl.ds(s * _ZSTRIPE, _ZSTRIPE)],
                        out.at[c].at[pl.ds(s * _ZSTRIPE, _ZSTRIPE)])

    return seg


_BN = 1000
_NBV = N_VARS // _BN


def _mlp_chain(h, wrefs, layers, n_relu):
    for i in range(len(layers)):
        w = wrefs[2 * i][...]
        b = wrefs[2 * i + 1][...]
        h = jnp.dot(h, w, preferred_element_type=_F32) + b
        if i < n_relu:
            h = jnp.maximum(h, 0.0)
    return h


def _stats_accumulate(st_ref, y, fw):
    s0 = jnp.sum(y, axis=0)
    s1 = jnp.sum(y * y, axis=0)
    row0 = jnp.pad(s0, (0, 128 - fw))[None, :]
    row1 = jnp.pad(s1, (0, 128 - fw))[None, :]
    upd = jnp.concatenate([row0, row1, jnp.zeros((6, 128), _F32)], axis=0)

    @pl.when(pl.program_id(0) == 0)
    def _():
        st_ref[...] = jnp.zeros_like(st_ref)

    st_ref[...] += upd


def _wspecs(layers):
    specs = []
    for (w, b) in layers:
        specs.append(pl.BlockSpec(w.shape, lambda i: (0, 0)))
        specs.append(pl.BlockSpec(b.shape, lambda i: (0, 0)))
    return specs


def _wargs(layers):
    out = []
    for (w, b) in layers:
        out.extend((w, b))
    return out


def _var_query_fused(variables, n1, layers):
    nw = 2 * len(layers)

    def body(*refs):
        var_ref, n_ref = refs[:2]
        wrefs = refs[2:2 + nw]
        lo_ref, hi_ref, sgn_ref = refs[2 + nw:]
        sgn = pl.program_id(0)
        sg = 1.0 - 2.0 * sgn.astype(_F32)
        x = jnp.concatenate([var_ref[...], n_ref[...]], axis=1)
        vq = _mlp_chain(x, wrefs, layers, 1)
        lit = jax.nn.softplus(sg * vq)
        lo_ref[...] = lit[:, :32]
        hi_ref[...] = lit[:, 32:]
        sgn_ref[...] = -sg * jax.nn.sigmoid(sg * vq)

    return pl.pallas_call(
        body,
        grid=(2, N_VARS // _BN),
        in_specs=[
            pl.BlockSpec((_BN, 64), lambda s, r: (r, 0)),
            pl.BlockSpec((_BN, 4), lambda s, r: (r, 0)),
        ] + [pl.BlockSpec(w.shape, lambda s, r: (0, 0))
             for (w, b) in layers for w in (w, b)],
        out_specs=[
            pl.BlockSpec((_BN, 32), lambda s, r: (s * _NBV + r, 0)),
            pl.BlockSpec((_BN, 32), lambda s, r: (s * _NBV + r, 0)),
            pl.BlockSpec((_BN, 64), lambda s, r: (s * _NBV + r, 0)),
        ],
        out_shape=[
            jax.ShapeDtypeStruct((N_LITS, 32), _F32),
            jax.ShapeDtypeStruct((N_LITS, 32), _F32),
            jax.ShapeDtypeStruct((N_LITS, 64), _F32),
        ],
    )(variables, n1, *_wargs(layers))


def _vars_out_fused(ug, st, vars_old, noise3, layers):
    mu, scale = _pn_scale_rows(st, N_VARS, FEATURE_MAPS)
    mu2 = jnp.broadcast_to(mu[None, :], (8, 64)).astype(_F32)
    sc2 = jnp.broadcast_to(scale[None, None], (8, 128)).astype(_F32)
    nw = 2 * len(layers)

    def body(*refs):
        x_ref, mu_ref, sc_ref, old_ref, nz_ref = refs[:5]
        wrefs = refs[5:5 + nw]
        nv_ref, lg_ref, l16_ref = refs[5 + nw:]
        sgn = pl.program_id(1)
        sg = 1.0 - 2.0 * sgn.astype(_F32)
        nv = (x_ref[...] - mu_ref[0:1, :]) * sc_ref[0, 0] * 0.25
        nv = nv + 0.1 * old_ref[...]
        nv_ref[...] = nv
        lg = _mlp_chain(nv, wrefs, layers, 1)[:, :1] + nz_ref[...]
        lg_ref[...] = lg
        l16_ref[...] = jnp.pad(jax.nn.softplus(sg * lg), ((0, 0), (0, 15)))

    return pl.pallas_call(
        body,
        grid=(N_VARS // _BN, 2),
        in_specs=[
            pl.BlockSpec((_BN, 64), lambda r, s: (r, 0)),
            pl.BlockSpec((8, 64), lambda r, s: (0, 0)),
            pl.BlockSpec((8, 128), lambda r, s: (0, 0)),
            pl.BlockSpec((_BN, 64), lambda r, s: (r, 0)),
            pl.BlockSpec((_BN, 1), lambda r, s: (r, 0)),
        ] + [pl.BlockSpec(w.shape, lambda r, s: (0, 0))
             for (w, b) in layers for w in (w, b)],
        out_specs=[
            pl.BlockSpec((_BN, 64), lambda r, s: (r, 0)),
            pl.BlockSpec((_BN, 1), lambda r, s: (r, 0)),
            pl.BlockSpec((_BN, 16), lambda r, s: (s * _NBV + r, 0)),
        ],
        out_shape=[
            jax.ShapeDtypeStruct((N_VARS, 64), _F32),
            jax.ShapeDtypeStruct((N_VARS, 1), _F32),
            jax.ShapeDtypeStruct((N_LITS, 16), _F32),
        ],
    )(ug, mu2, sc2, vars_old, noise3, *_wargs(layers))


def _clause_mlp_fused(s_sc, cq, cs, mask, layers):
    nw = 2 * len(layers)

    def body(*refs):
        (slo, shi, cq_ref, cs_ref, m_ref) = refs[:5]
        wrefs = refs[5:5 + nw]
        cd_ref, st_ref, cl_lo_ref, cl_hi_ref = refs[5 + nw:]
        s_full = jnp.concatenate([slo[0], shi[0]], axis=1)
        e = jnp.exp(-s_full)
        cl = e * cq_ref[...]
        cu = jnp.concatenate([cs_ref[...], 4.0 * cl, e], axis=1) * m_ref[...]
        h = _mlp_chain(cu, wrefs, layers, 1)
        cd_ref[...] = h
        _stats_accumulate(st_ref, h[:, 64:], 64)
        cl_lo_ref[...] = cl[:, :32]
        cl_hi_ref[...] = cl[:, 32:]

    return pl.pallas_call(
        body,
        grid=(N_CLAUSES // _BN,),
        in_specs=[
            pl.BlockSpec((1, _BN, 32), lambda r: (0, r, 0)),
            pl.BlockSpec((1, _BN, 32), lambda r: (1, r, 0)),
            pl.BlockSpec((_BN, 64), lambda r: (r, 0)),
            pl.BlockSpec((_BN, 64), lambda r: (r, 0)),
            pl.BlockSpec((_BN, 1), lambda r: (r, 0)),
        ] + _wspecs(layers),
        out_specs=[
            pl.BlockSpec((_BN, 128), lambda r: (r, 0)),
            pl.BlockSpec((8, 128), lambda r: (0, 0)),
            pl.BlockSpec((_BN, 32), lambda r: (r, 0)),
            pl.BlockSpec((_BN, 32), lambda r: (r, 0)),
        ],
        out_shape=[
            jax.ShapeDtypeStruct((N_CLAUSES, 128), _F32),
            jax.ShapeDtypeStruct((8, 128), _F32),
            jax.ShapeDtypeStruct((N_CLAUSES, 32), _F32),
            jax.ShapeDtypeStruct((N_CLAUSES, 32), _F32),
        ],
        compiler_params=pltpu.CompilerParams(
            dimension_semantics=("arbitrary",)),
    )(s_sc, s_sc, cq, cs, mask, *_wargs(layers))


def _update_gate_fused(sgn, h_sc, vl_sc, vdw, dw, variables, layers):
    nw = 2 * len(layers)

    def body(*refs):
        (sg0, sg1, h00, h10, h01, h11, v00, v10, v01, v11,
         vdw_ref, dw0, dw1, var_ref) = refs[:14]
        wrefs = refs[14:14 + nw]
        out_ref, st_ref = refs[14 + nw:]
        h_lo = jnp.concatenate([h00[0], h10[0]], axis=1)
        h_hi = jnp.concatenate([h01[0], h11[0]], axis=1)
        vgrad = (sg0[...] * h_lo + sg1[...] * h_hi) * vdw_ref[...]
        vl_lo = jnp.concatenate([v00[0], v10[0]], axis=1) * dw0[...]
        vl_hi = jnp.concatenate([v01[0], v11[0]], axis=1) * dw1[...]
        unit = jnp.concatenate(
            [vgrad, var_ref[...], vl_lo, vl_hi], axis=1)
        h = _mlp_chain(unit, wrefs, layers, 2)
        out_ref[...] = h
        _stats_accumulate(st_ref, h, 64)

    nb = N_VARS // _BN
    return pl.pallas_call(
        body,
        grid=(nb,),
        in_specs=[
            pl.BlockSpec((_BN, 64), lambda r: (r, 0)),
            pl.BlockSpec((_BN, 64), lambda r: (_NBV + r, 0)),
            pl.BlockSpec((1, _BN, 32), lambda r: (0, r, 0)),
            pl.BlockSpec((1, _BN, 32), lambda r: (1, r, 0)),
            pl.BlockSpec((1, _BN, 32), lambda r: (0, _NBV + r, 0)),
            pl.BlockSpec((1, _BN, 32), lambda r: (1, _NBV + r, 0)),
            pl.BlockSpec((1, _BN, 32), lambda r: (0, r, 0)),
            pl.BlockSpec((1, _BN, 32), lambda r: (1, r, 0)),
            pl.BlockSpec((1, _BN, 32), lambda r: (0, _NBV + r, 0)),
            pl.BlockSpec((1, _BN, 32), lambda r: (1, _NBV + r, 0)),
            pl.BlockSpec((_BN, 1), lambda r: (r, 0)),
            pl.BlockSpec((_BN, 1), lambda r: (r, 0)),
            pl.BlockSpec((_BN, 1), lambda r: (_NBV + r, 0)),
            pl.BlockSpec((_BN, 64), lambda r: (r, 0)),
        ] + _wspecs(layers),
        out_specs=[
            pl.BlockSpec((_BN, 64), lambda r: (r, 0)),
            pl.BlockSpec((8, 128), lambda r: (0, 0)),
        ],
        out_shape=[
            jax.ShapeDtypeStruct((N_VARS, 64), _F32),
            jax.ShapeDtypeStruct((8, 128), _F32),
        ],
        compiler_params=pltpu.CompilerParams(
            dimension_semantics=("arbitrary",)),
    )(sgn, sgn, h_sc, h_sc, h_sc, h_sc, vl_sc, vl_sc, vl_sc, vl_sc,
      vdw, dw, dw, variables, *_wargs(layers))


def _mlp_pallas(xs, layers, n_relu, stats_lo=None, stats_hi=None):
    n = xs[0].shape[0]
    dout = layers[-1][0].shape[1]
    grid = n // _BN
    nx = len(xs)
    with_stats = stats_lo is not None

    def body(*refs):
        nl = len(layers)
        out_ref = refs[nx + 2 * nl]
        h = jnp.concatenate([r[...] for r in refs[:nx]], axis=1)
        for i in range(nl):
            w = refs[nx + 2 * i][...]
            b = refs[nx + 2 * i + 1][...]
            h = jnp.dot(h, w, preferred_element_type=_F32) + b
            if i < n_relu:
                h = jnp.maximum(h, 0.0)
        out_ref[...] = h
        if with_stats:
            st_ref = refs[nx + 2 * nl + 1]
            y = h[:, stats_lo:stats_hi]
            fw = stats_hi - stats_lo
            s0 = jnp.sum(y, axis=0)
            s1 = jnp.sum(y * y, axis=0)
            row0 = jnp.pad(s0, (0, 128 - fw))[None, :]
            row1 = jnp.pad(s1, (0, 128 - fw))[None, :]
            upd = jnp.concatenate(
                [row0, row1, jnp.zeros((6, 128), _F32)], axis=0)

            @pl.when(pl.program_id(0) == 0)
            def _():
                st_ref[...] = jnp.zeros_like(st_ref)

            st_ref[...] += upd

    in_specs = [pl.BlockSpec((_BN, x.shape[1]), lambda i: (i, 0)) for x in xs]
    args = list(xs)
    for (w, b) in layers:
        in_specs.append(pl.BlockSpec(w.shape, lambda i: (0, 0)))
        in_specs.append(pl.BlockSpec(b.shape, lambda i: (0, 0)))
        args.extend((w, b))
    out_shape = [jax.ShapeDtypeStruct((n, dout), _F32)]
    out_specs = [pl.BlockSpec((_BN, dout), lambda i: (i, 0))]
    if with_stats:
        out_shape.append(jax.ShapeDtypeStruct((8, 128), _F32))
        out_specs.append(pl.BlockSpec((8, 128), lambda i: (0, 0)))
    res = pl.pallas_call(
        body,
        grid=(grid,),
        in_specs=in_specs,
        out_specs=out_specs,
        out_shape=out_shape,
        compiler_params=pltpu.CompilerParams(
            dimension_semantics=("arbitrary",)),
    )(*args)
    return res if with_stats else res[0]


def _pn_scale_rows(stats, n, fw, eps=1e-6):
    s0 = stats[0, :fw]
    s1 = stats[1, :fw]
    mu = s0 / np.float32(n)
    var = (jnp.sum(s1) - np.float32(n) * jnp.sum(mu * mu)) / np.float32(n * fw)
    scale = lax.rsqrt(var + np.float32(eps))
    return mu, scale


def _clause_state_update(cd, st, cs_old):
    mu, scale = _pn_scale_rows(st, N_CLAUSES, FEATURE_MAPS)
    mu2 = jnp.broadcast_to(mu[None, :], (8, 64)).astype(_F32)
    sc2 = jnp.broadcast_to(scale[None, None], (8, 128)).astype(_F32)

    def body(cd_ref, mu_ref, sc_ref, cs_ref, out_ref, vlo_ref, vhi_ref):
        d = cd_ref[...]
        x = (d[:, 64:] - mu_ref[0:1, :]) * sc_ref[0, 0] * 0.25
        out_ref[...] = x + 0.1 * cs_ref[...]
        vlo_ref[...] = d[:, :32]
        vhi_ref[...] = d[:, 32:64]

    return pl.pallas_call(
        body,
        grid=(N_CLAUSES // _BN,),
        in_specs=[
            pl.BlockSpec((_BN, 128), lambda r: (r, 0)),
            pl.BlockSpec((8, 64), lambda r: (0, 0)),
            pl.BlockSpec((8, 128), lambda r: (0, 0)),
            pl.BlockSpec((_BN, 64), lambda r: (r, 0)),
        ],
        out_specs=[
            pl.BlockSpec((_BN, 64), lambda r: (r, 0)),
            pl.BlockSpec((_BN, 32), lambda r: (r, 0)),
            pl.BlockSpec((_BN, 32), lambda r: (r, 0)),
        ],
        out_shape=[
            jax.ShapeDtypeStruct((N_CLAUSES, 64), _F32),
            jax.ShapeDtypeStruct((N_CLAUSES, 32), _F32),
            jax.ShapeDtypeStruct((N_CLAUSES, 32), _F32),
        ],
    )(cd, mu2, sc2, cs_old)


def _prep_layers(params_list):
    return [(w, b[None, :]) for (w, b) in params_list]


def kernel(edge_index, clauses_mask_sigmoid, clauses_graph, variables_graph, params):
    row = edge_index[0]
    col = edge_index[1]
    pad = _EPAD - N_EDGES
    zpad = jnp.zeros((pad,), _I32)
    dpad = jnp.full((pad,), _DUMP, _I32)
    row_g = jnp.concatenate([row, zpad])
    col_g = jnp.concatenate([col, zpad])
    row_s = jnp.concatenate([row, dpad])
    col_s = jnp.concatenate([col, dpad])
    zeros32 = jnp.zeros((_ZSTRIPE, 32), _F32)
    zeros16 = jnp.zeros((_ZSTRIPE, 16), _F32)
    seg_w = _make_seg_kernel(32, True)
    seg_n = _make_seg_kernel(16, False)

    clauses_mask = clauses_mask_sigmoid[:, None]

    ones16 = jnp.zeros((_NROWS, 16), _F32).at[:, 0].set(1.0)
    degp = seg_n(ones16, ones16, row_g, row_s, zeros16)
    lit_degree = (degp[0, :_NROWS, 0] + degp[1, :_NROWS, 0])[:, None]
    degree_weight = lax.rsqrt(jnp.maximum(lit_degree, 1.0))
    var_degree_weight = 4.0 * lax.rsqrt(
        jnp.maximum(lit_degree[:N_VARS] + lit_degree[N_VARS:], 1.0))

    p_vq = _prep_layers(params['variables_query'])
    p_cq = _prep_layers(params['clauses_query'])
    p_cm = _prep_layers(params['clause_mlp'])
    p_ug = _prep_layers(params['update_gate'])
    p_vo = _prep_layers(params['variables_output'])
    w_last, b_last = p_vo[1]
    p_vo = [p_vo[0], (jnp.pad(w_last, ((0, 0), (0, 127))),
                      jnp.pad(b_last, ((0, 0), (0, 127))))]

    variables = jnp.ones((N_VARS, FEATURE_MAPS), _F32)
    clause_state = jnp.ones((N_CLAUSES, FEATURE_MAPS), _F32)
    last_logits = jnp.zeros((N_VARS, 1), _F32)
    step_losses = []
    for step in range(ROUNDS):
        with jax.ensure_compile_time_eval():
            rkey = jax.random.key(42)
            k1 = jax.random.fold_in(rkey, 3 * step)
            k2 = jax.random.fold_in(rkey, 3 * step + 1)
            k3 = jax.random.fold_in(rkey, 3 * step + 2)
            n1 = jax.random.normal(k1, (N_VARS, 4), _F32)
            n2 = jax.random.normal(k2, (N_CLAUSES, 4), _F32)
            u3 = jax.random.uniform(k3, (N_VARS, 1),
                                    minval=1e-5, maxval=1.0 - 1e-5)
            noise3 = jnp.log(u3 / (1.0 - u3))
        clause_query = _mlp_pallas([clause_state, clauses_mask, n2], p_cq, 1)
        lit_lo, lit_hi, sgn_tab = _var_query_fused(variables, n1, p_vq)
        s_sc = seg_w(lit_lo, lit_hi, row_g, col_s, zeros32)

        clause_data, st_c, cl_lo, cl_hi = _clause_mlp_fused(
            s_sc, clause_query, clause_state, clauses_mask, p_cm)
        h_sc = seg_w(cl_lo, cl_hi, col_g, row_s, zeros32)

        clause_state, vl_lo, vl_hi = _clause_state_update(
            clause_data, st_c, clause_state)

        vl_sc = seg_w(vl_lo, vl_hi, col_g, row_s, zeros32)
        ug_out, st_v = _update_gate_fused(
            sgn_tab, h_sc, vl_sc, var_degree_weight, degree_weight,
            variables, p_ug)
        variables, logits, lit1_16 = _vars_out_fused(
            ug_out, st_v, variables, noise3, p_vo)

        s1p = seg_n(lit1_16, lit1_16, row_g, col_s, zeros16)
        s1 = (s1p[0, :_NROWS, 0] + s1p[1, :_NROWS, 0])[:, None]
        clauses_val = jnp.exp(-s1) * clauses_mask
        per_clause = clauses_val * -jnp.log(1.0 - clauses_val + 1e-10)
        per_graph = clauses_graph @ per_clause
        step_losses.append(
            jnp.sqrt(per_graph + 1e-6) - np.float32(np.sqrt(1e-6)))
        last_logits = logits
    unsupervised_loss = sum(step_losses) / np.float32(ROUNDS)
    return last_logits, unsupervised_loss, jnp.array(ROUNDS - 1, jnp.int32)


# speedup vs baseline: 1.0024x; 1.0024x over previous
import functools

import jax
import jax.numpy as jnp
import numpy as np
from jax import lax
from jax.experimental import pallas as pl
from jax.experimental.pallas import tpu as pltpu
from jax.experimental.pallas import tpu_sc as plsc

N_VARS = 25000
N_CLAUSES = 50000
N_LITS = 50000
N_EDGES = 800000
FEATURE_MAPS = 64
QUERY_MAPS = 64
ROUNDS = 4

_F32 = jnp.float32
_I32 = jnp.int32


_NC, _NS = 2, 16
_EPAD = 802816
_NROWS = 50000
_NDUMP = 50048
_ZSTRIPE = _NDUMP // _NS
_DUMP = _NROWS


@functools.lru_cache(maxsize=None)
def _make_seg_kernel(fh, split_features):
    chunk = 392 if split_features else 896
    per_tile = _EPAD // _NS if split_features else _EPAD // (_NC * _NS)
    nchunks = per_tile // chunk
    mesh = plsc.VectorSubcoreMesh(
        core_axis_name="c", subcore_axis_name="s",
        num_cores=_NC, num_subcores=_NS)

    @functools.partial(
        pl.kernel,
        out_type=jax.ShapeDtypeStruct((_NC, _NDUMP, fh), _F32),
        mesh=mesh,
        scratch_types=(
            [pltpu.VMEM_SHARED((_NDUMP, fh), _F32)]
            + [pltpu.VMEM((chunk, fh), _F32)] * 2
            + [pltpu.VMEM((chunk,), _I32)] * 8
            + [pltpu.SemaphoreType.DMA] * 3
        ),
        compiler_params=pltpu.CompilerParams(use_tc_tiling_on_sc=False),
    )
    def seg(tabA, tabB, gidx, sidx, zeros, out, acc, gbuf0, gbuf1,
            giv0, giv1, giv2, giv3, siv0, siv1, siv2, siv3,
            gsem, ssem, isem):
        c = lax.axis_index("c")
        s = lax.axis_index("s")
        pltpu.sync_copy(zeros, acc.at[pl.ds(s * _ZSTRIPE, _ZSTRIPE)])
        plsc.subcore_barrier()
        if split_features:
            base = s * per_tile
        else:
            base = (c * _NS + s) * per_tile
        gbufs = (gbuf0, gbuf1)
        givs = (giv0, giv1, giv2, giv3)
        sivs = (siv0, siv1, siv2, siv3)
        dummy = tabA.at[pl.ds(0, chunk)]
        idummy = gidx.at[pl.ds(0, chunk)]

        def idx_start(i, r):
            off = base + i * chunk
            pltpu.async_copy(gidx.at[pl.ds(off, chunk)], givs[r], isem)
            pltpu.async_copy(sidx.at[pl.ds(off, chunk)], sivs[r], isem)

        def iwait(r):
            pltpu.make_async_copy(idummy, givs[r], isem).wait()
            pltpu.make_async_copy(idummy, sivs[r], isem).wait()

        def gather_start(r, b):
            if split_features:
                @pl.when(c == 0)
                def _():
                    pltpu.async_copy(tabA.at[givs[r]], gbufs[b], gsem)

                @pl.when(c == 1)
                def _():
                    pltpu.async_copy(tabB.at[givs[r]], gbufs[b], gsem)
            else:
                pltpu.async_copy(tabA.at[givs[r]], gbufs[b], gsem)

        def gwait(b):
            pltpu.make_async_copy(dummy, gbufs[b], gsem).wait()

        def swait(b):
            pltpu.make_async_copy(dummy, gbufs[b], ssem).wait()

        idx_start(0, 0)
        idx_start(1, 1)
        iwait(0)
        gather_start(0, 0)

        def group(g, carry):
            for sub in range(4):
                i = 4 * g + sub
                b = sub % 2
                gwait(b)
                pltpu.async_copy(gbufs[b], acc.at[sivs[sub]], ssem, add=True)

                @pl.when(i >= 1)
                def _():
                    swait(1 - b)

                @pl.when(i + 2 < nchunks)
                def _():
                    idx_start(i + 2, (sub + 2) % 4)

                @pl.when(i + 1 < nchunks)
                def _():
                    iwait((sub + 1) % 4)
                    gather_start((sub + 1) % 4, 1 - b)
            return carry

        lax.fori_loop(0, nchunks // 4, group, 0)
        swait((nchunks - 1) % 2)
        plsc.subcore_barrier()
        pltpu.sync_copy(acc.at[pl.ds(s * _ZSTRIPE, _ZSTRIPE)],
                        out.at[c].at[pl.ds(s * _ZSTRIPE, _ZSTRIPE)])

    return seg


_BN = 1000
_NBV = N_VARS // _BN


def _mlp_chain(h, wrefs, layers, n_relu):
    for i in range(len(layers)):
        w = wrefs[2 * i][...]
        b = wrefs[2 * i + 1][...]
        h = jnp.dot(h, w, preferred_element_type=_F32) + b
        if i < n_relu:
            h = jnp.maximum(h, 0.0)
    return h


def _stats_accumulate(st_ref, y, fw):
    s0 = jnp.sum(y, axis=0)
    s1 = jnp.sum(y * y, axis=0)
    row0 = jnp.pad(s0, (0, 128 - fw))[None, :]
    row1 = jnp.pad(s1, (0, 128 - fw))[None, :]
    upd = jnp.concatenate([row0, row1, jnp.zeros((6, 128), _F32)], axis=0)

    @pl.when(pl.program_id(0) == 0)
    def _():
        st_ref[...] = jnp.zeros_like(st_ref)

    st_ref[...] += upd


def _wspecs(layers):
    specs = []
    for (w, b) in layers:
        specs.append(pl.BlockSpec(w.shape, lambda i: (0, 0)))
        specs.append(pl.BlockSpec(b.shape, lambda i: (0, 0)))
    return specs


def _wargs(layers):
    out = []
    for (w, b) in layers:
        out.extend((w, b))
    return out


def _clause_mlp_fused(s_sc, cq, cs, mask, layers):
    nw = 2 * len(layers)

    def body(*refs):
        (slo, shi, cq_ref, cs_ref, m_ref) = refs[:5]
        wrefs = refs[5:5 + nw]
        cd_ref, st_ref, cl_lo_ref, cl_hi_ref = refs[5 + nw:]
        s_full = jnp.concatenate([slo[0], shi[0]], axis=1)
        e = jnp.exp(-s_full)
        cl = e * cq_ref[...]
        cu = jnp.concatenate([cs_ref[...], 4.0 * cl, e], axis=1) * m_ref[...]
        h = _mlp_chain(cu, wrefs, layers, 1)
        cd_ref[...] = h
        _stats_accumulate(st_ref, h[:, 64:], 64)
        cl_lo_ref[...] = cl[:, :32]
        cl_hi_ref[...] = cl[:, 32:]

    return pl.pallas_call(
        body,
        grid=(N_CLAUSES // _BN,),
        in_specs=[
            pl.BlockSpec((1, _BN, 32), lambda r: (0, r, 0)),
            pl.BlockSpec((1, _BN, 32), lambda r: (1, r, 0)),
            pl.BlockSpec((_BN, 64), lambda r: (r, 0)),
            pl.BlockSpec((_BN, 64), lambda r: (r, 0)),
            pl.BlockSpec((_BN, 1), lambda r: (r, 0)),
        ] + _wspecs(layers),
        out_specs=[
            pl.BlockSpec((_BN, 128), lambda r: (r, 0)),
            pl.BlockSpec((8, 128), lambda r: (0, 0)),
            pl.BlockSpec((_BN, 32), lambda r: (r, 0)),
            pl.BlockSpec((_BN, 32), lambda r: (r, 0)),
        ],
        out_shape=[
            jax.ShapeDtypeStruct((N_CLAUSES, 128), _F32),
            jax.ShapeDtypeStruct((8, 128), _F32),
            jax.ShapeDtypeStruct((N_CLAUSES, 32), _F32),
            jax.ShapeDtypeStruct((N_CLAUSES, 32), _F32),
        ],
        compiler_params=pltpu.CompilerParams(
            dimension_semantics=("arbitrary",)),
    )(s_sc, s_sc, cq, cs, mask, *_wargs(layers))


def _update_gate_fused(sgn, h_sc, vl_sc, vdw, dw, variables, layers):
    nw = 2 * len(layers)

    def body(*refs):
        (sg0, sg1, h00, h10, h01, h11, v00, v10, v01, v11,
         vdw_ref, dw0, dw1, var_ref) = refs[:14]
        wrefs = refs[14:14 + nw]
        out_ref, st_ref = refs[14 + nw:]
        h_lo = jnp.concatenate([h00[0], h10[0]], axis=1)
        h_hi = jnp.concatenate([h01[0], h11[0]], axis=1)
        vgrad = (sg0[...] * h_lo + sg1[...] * h_hi) * vdw_ref[...]
        vl_lo = jnp.concatenate([v00[0], v10[0]], axis=1) * dw0[...]
        vl_hi = jnp.concatenate([v01[0], v11[0]], axis=1) * dw1[...]
        unit = jnp.concatenate(
            [vgrad, var_ref[...], vl_lo, vl_hi], axis=1)
        h = _mlp_chain(unit, wrefs, layers, 2)
        out_ref[...] = h
        _stats_accumulate(st_ref, h, 64)

    nb = N_VARS // _BN
    return pl.pallas_call(
        body,
        grid=(nb,),
        in_specs=[
            pl.BlockSpec((_BN, 64), lambda r: (r, 0)),
            pl.BlockSpec((_BN, 64), lambda r: (_NBV + r, 0)),
            pl.BlockSpec((1, _BN, 32), lambda r: (0, r, 0)),
            pl.BlockSpec((1, _BN, 32), lambda r: (1, r, 0)),
            pl.BlockSpec((1, _BN, 32), lambda r: (0, _NBV + r, 0)),
            pl.BlockSpec((1, _BN, 32), lambda r: (1, _NBV + r, 0)),
            pl.BlockSpec((1, _BN, 32), lambda r: (0, r, 0)),
            pl.BlockSpec((1, _BN, 32), lambda r: (1, r, 0)),
            pl.BlockSpec((1, _BN, 32), lambda r: (0, _NBV + r, 0)),
            pl.BlockSpec((1, _BN, 32), lambda r: (1, _NBV + r, 0)),
            pl.BlockSpec((_BN, 1), lambda r: (r, 0)),
            pl.BlockSpec((_BN, 1), lambda r: (r, 0)),
            pl.BlockSpec((_BN, 1), lambda r: (_NBV + r, 0)),
            pl.BlockSpec((_BN, 64), lambda r: (r, 0)),
        ] + _wspecs(layers),
        out_specs=[
            pl.BlockSpec((_BN, 64), lambda r: (r, 0)),
            pl.BlockSpec((8, 128), lambda r: (0, 0)),
        ],
        out_shape=[
            jax.ShapeDtypeStruct((N_VARS, 64), _F32),
            jax.ShapeDtypeStruct((8, 128), _F32),
        ],
        compiler_params=pltpu.CompilerParams(
            dimension_semantics=("arbitrary",)),
    )(sgn, sgn, h_sc, h_sc, h_sc, h_sc, vl_sc, vl_sc, vl_sc, vl_sc,
      vdw, dw, dw, variables, *_wargs(layers))


def _mlp_pallas(xs, layers, n_relu, stats_lo=None, stats_hi=None):
    n = xs[0].shape[0]
    dout = layers[-1][0].shape[1]
    grid = n // _BN
    nx = len(xs)
    with_stats = stats_lo is not None

    def body(*refs):
        nl = len(layers)
        out_ref = refs[nx + 2 * nl]
        h = jnp.concatenate([r[...] for r in refs[:nx]], axis=1)
        for i in range(nl):
            w = refs[nx + 2 * i][...]
            b = refs[nx + 2 * i + 1][...]
            h = jnp.dot(h, w, preferred_element_type=_F32) + b
            if i < n_relu:
                h = jnp.maximum(h, 0.0)
        out_ref[...] = h
        if with_stats:
            st_ref = refs[nx + 2 * nl + 1]
            y = h[:, stats_lo:stats_hi]
            fw = stats_hi - stats_lo
            s0 = jnp.sum(y, axis=0)
            s1 = jnp.sum(y * y, axis=0)
            row0 = jnp.pad(s0, (0, 128 - fw))[None, :]
            row1 = jnp.pad(s1, (0, 128 - fw))[None, :]
            upd = jnp.concatenate(
                [row0, row1, jnp.zeros((6, 128), _F32)], axis=0)

            @pl.when(pl.program_id(0) == 0)
            def _():
                st_ref[...] = jnp.zeros_like(st_ref)

            st_ref[...] += upd

    in_specs = [pl.BlockSpec((_BN, x.shape[1]), lambda i: (i, 0)) for x in xs]
    args = list(xs)
    for (w, b) in layers:
        in_specs.append(pl.BlockSpec(w.shape, lambda i: (0, 0)))
        in_specs.append(pl.BlockSpec(b.shape, lambda i: (0, 0)))
        args.extend((w, b))
    out_shape = [jax.ShapeDtypeStruct((n, dout), _F32)]
    out_specs = [pl.BlockSpec((_BN, dout), lambda i: (i, 0))]
    if with_stats:
        out_shape.append(jax.ShapeDtypeStruct((8, 128), _F32))
        out_specs.append(pl.BlockSpec((8, 128), lambda i: (0, 0)))
    res = pl.pallas_call(
        body,
        grid=(grid,),
        in_specs=in_specs,
        out_specs=out_specs,
        out_shape=out_shape,
        compiler_params=pltpu.CompilerParams(
            dimension_semantics=("arbitrary",)),
    )(*args)
    return res if with_stats else res[0]


def _lit_table(vq):
    def body(x_ref, lo_ref, hi_ref):
        sgn = pl.program_id(0)
        sg = 1.0 - 2.0 * sgn.astype(_F32)
        v = jax.nn.softplus(sg * x_ref[...])
        lo_ref[...] = v[:, :32]
        hi_ref[...] = v[:, 32:]

    return pl.pallas_call(
        body,
        grid=(2, N_VARS // _BN),
        in_specs=[pl.BlockSpec((_BN, 64), lambda s, r: (r, 0))],
        out_specs=[
            pl.BlockSpec((_BN, 32), lambda s, r: (s * _NBV + r, 0)),
            pl.BlockSpec((_BN, 32), lambda s, r: (s * _NBV + r, 0)),
        ],
        out_shape=[
            jax.ShapeDtypeStruct((N_LITS, 32), _F32),
            jax.ShapeDtypeStruct((N_LITS, 32), _F32),
        ],
    )(vq)


def _sgn_table(vq):
    def body(x_ref, o_ref):
        sgn = pl.program_id(0)
        sg = 2.0 * sgn.astype(_F32) - 1.0
        o_ref[...] = sg * jax.nn.sigmoid(-sg * x_ref[...])

    return pl.pallas_call(
        body,
        grid=(2, N_VARS // _BN),
        in_specs=[pl.BlockSpec((_BN, 64), lambda s, r: (r, 0))],
        out_specs=pl.BlockSpec((_BN, 64), lambda s, r: (s * _NBV + r, 0)),
        out_shape=jax.ShapeDtypeStruct((N_LITS, 64), _F32),
    )(vq)


def _pn_scale_rows(stats, n, fw, eps=1e-6):
    s0 = stats[0, :fw]
    s1 = stats[1, :fw]
    mu = s0 / np.float32(n)
    var = (jnp.sum(s1) - np.float32(n) * jnp.sum(mu * mu)) / np.float32(n * fw)
    scale = lax.rsqrt(var + np.float32(eps))
    return mu, scale


def _clause_state_update(cd, st, cs_old):
    mu, scale = _pn_scale_rows(st, N_CLAUSES, FEATURE_MAPS)
    mu2 = jnp.broadcast_to(mu[None, :], (8, 64)).astype(_F32)
    sc2 = jnp.broadcast_to(scale[None, None], (8, 128)).astype(_F32)

    def body(cd_ref, mu_ref, sc_ref, cs_ref, out_ref, vlo_ref, vhi_ref):
        d = cd_ref[...]
        x = (d[:, 64:] - mu_ref[0:1, :]) * sc_ref[0, 0] * 0.25
        out_ref[...] = x + 0.1 * cs_ref[...]
        vlo_ref[...] = d[:, :32]
        vhi_ref[...] = d[:, 32:64]

    return pl.pallas_call(
        body,
        grid=(N_CLAUSES // _BN,),
        in_specs=[
            pl.BlockSpec((_BN, 128), lambda r: (r, 0)),
            pl.BlockSpec((8, 64), lambda r: (0, 0)),
            pl.BlockSpec((8, 128), lambda r: (0, 0)),
            pl.BlockSpec((_BN, 64), lambda r: (r, 0)),
        ],
        out_specs=[
            pl.BlockSpec((_BN, 64), lambda r: (r, 0)),
            pl.BlockSpec((_BN, 32), lambda r: (r, 0)),
            pl.BlockSpec((_BN, 32), lambda r: (r, 0)),
        ],
        out_shape=[
            jax.ShapeDtypeStruct((N_CLAUSES, 64), _F32),
            jax.ShapeDtypeStruct((N_CLAUSES, 32), _F32),
            jax.ShapeDtypeStruct((N_CLAUSES, 32), _F32),
        ],
    )(cd, mu2, sc2, cs_old)


def _vars_update(ug, st, vars_old):
    mu, scale = _pn_scale_rows(st, N_VARS, FEATURE_MAPS)
    mu2 = jnp.broadcast_to(mu[None, :], (8, 64)).astype(_F32)
    sc2 = jnp.broadcast_to(scale[None, None], (8, 128)).astype(_F32)

    def body(x_ref, mu_ref, sc_ref, old_ref, out_ref):
        x = (x_ref[...] - mu_ref[0:1, :]) * sc_ref[0, 0] * 0.25
        out_ref[...] = x + 0.1 * old_ref[...]

    return pl.pallas_call(
        body,
        grid=(N_VARS // _BN,),
        in_specs=[
            pl.BlockSpec((_BN, 64), lambda r: (r, 0)),
            pl.BlockSpec((8, 64), lambda r: (0, 0)),
            pl.BlockSpec((8, 128), lambda r: (0, 0)),
            pl.BlockSpec((_BN, 64), lambda r: (r, 0)),
        ],
        out_specs=pl.BlockSpec((_BN, 64), lambda r: (r, 0)),
        out_shape=jax.ShapeDtypeStruct((N_VARS, 64), _F32),
    )(ug, mu2, sc2, vars_old)


def _lit1_table(logits_n):
    def body(x_ref, o_ref):
        sgn = pl.program_id(0)
        sg = 1.0 - 2.0 * sgn.astype(_F32)
        v = jax.nn.softplus(sg * x_ref[...])
        o_ref[...] = jnp.pad(v, ((0, 0), (0, 15)))

    return pl.pallas_call(
        body,
        grid=(2, N_VARS // _BN),
        in_specs=[pl.BlockSpec((_BN, 1), lambda s, r: (r, 0))],
        out_specs=pl.BlockSpec((_BN, 16), lambda s, r: (s * _NBV + r, 0)),
        out_shape=jax.ShapeDtypeStruct((N_LITS, 16), _F32),
    )(logits_n)


def _prep_layers(params_list):
    return [(w, b[None, :]) for (w, b) in params_list]


def kernel(edge_index, clauses_mask_sigmoid, clauses_graph, variables_graph, params):
    row = edge_index[0]
    col = edge_index[1]
    pad = _EPAD - N_EDGES
    zpad = jnp.zeros((pad,), _I32)
    dpad = jnp.full((pad,), _DUMP, _I32)
    row_g = jnp.concatenate([row, zpad])
    col_g = jnp.concatenate([col, zpad])
    row_s = jnp.concatenate([row, dpad])
    col_s = jnp.concatenate([col, dpad])
    zeros32 = jnp.zeros((_ZSTRIPE, 32), _F32)
    zeros16 = jnp.zeros((_ZSTRIPE, 16), _F32)
    seg_w = _make_seg_kernel(32, True)
    seg_n = _make_seg_kernel(16, False)

    clauses_mask = clauses_mask_sigmoid[:, None]

    ones16 = jnp.zeros((_NROWS, 16), _F32).at[:, 0].set(1.0)
    degp = seg_n(ones16, ones16, row_g, row_s, zeros16)
    lit_degree = (degp[0, :_NROWS, 0] + degp[1, :_NROWS, 0])[:, None]
    degree_weight = lax.rsqrt(jnp.maximum(lit_degree, 1.0))
    var_degree_weight = 4.0 * lax.rsqrt(
        jnp.maximum(lit_degree[:N_VARS] + lit_degree[N_VARS:], 1.0))

    p_vq = _prep_layers(params['variables_query'])
    p_cq = _prep_layers(params['clauses_query'])
    p_cm = _prep_layers(params['clause_mlp'])
    p_ug = _prep_layers(params['update_gate'])
    p_vo = _prep_layers(params['variables_output'])
    w_last, b_last = p_vo[1]
    p_vo = [p_vo[0], (jnp.pad(w_last, ((0, 0), (0, 127))),
                      jnp.pad(b_last, ((0, 0), (0, 127))))]

    variables = jnp.ones((N_VARS, FEATURE_MAPS), _F32)
    clause_state = jnp.ones((N_CLAUSES, FEATURE_MAPS), _F32)
    last_logits = jnp.zeros((N_VARS, 1), _F32)
    step_losses = []
    for step in range(ROUNDS):
        with jax.ensure_compile_time_eval():
            rkey = jax.random.key(42)
            k1 = jax.random.fold_in(rkey, 3 * step)
            k2 = jax.random.fold_in(rkey, 3 * step + 1)
            k3 = jax.random.fold_in(rkey, 3 * step + 2)
            n1 = jax.random.normal(k1, (N_VARS, 4), _F32)
            n2 = jax.random.normal(k2, (N_CLAUSES, 4), _F32)
            u3 = jax.random.uniform(k3, (N_VARS, 1),
                                    minval=1e-5, maxval=1.0 - 1e-5)
            noise3 = jnp.log(u3 / (1.0 - u3))
        var_query = _mlp_pallas([variables, n1], p_vq, 1)
        clause_query = _mlp_pallas([clause_state, clauses_mask, n2], p_cq, 1)

        lit_lo, lit_hi = _lit_table(var_query)
        s_sc = seg_w(lit_lo, lit_hi, row_g, col_s, zeros32)
        sgn_tab = _sgn_table(var_query)

        clause_data, st_c, cl_lo, cl_hi = _clause_mlp_fused(
            s_sc, clause_query, clause_state, clauses_mask, p_cm)
        h_sc = seg_w(cl_lo, cl_hi, col_g, row_s, zeros32)

        clause_state, vl_lo, vl_hi = _clause_state_update(
            clause_data, st_c, clause_state)

        vl_sc = seg_w(vl_lo, vl_hi, col_g, row_s, zeros32)
        ug_out, st_v = _update_gate_fused(
            sgn_tab, h_sc, vl_sc, var_degree_weight, degree_weight,
            variables, p_ug)
        variables = _vars_update(ug_out, st_v, variables)

        logits = _mlp_pallas([variables], p_vo, 1)[:, :1]
        logits = logits + noise3

        lit1_16 = _lit1_table(logits)
        s1p = seg_n(lit1_16, lit1_16, row_g, col_s, zeros16)
        s1 = (s1p[0, :_NROWS, 0] + s1p[1, :_NROWS, 0])[:, None]
        clauses_val = jnp.exp(-s1) * clauses_mask
        per_clause = clauses_val * -jnp.log(1.0 - clauses_val + 1e-10)
        per_graph = clauses_graph @ per_clause
        step_losses.append(
            jnp.sqrt(per_graph + 1e-6) - np.float32(np.sqrt(1e-6)))
        last_logits = logits
    unsupervised_loss = sum(step_losses) / np.float32(ROUNDS)
    return last_logits, unsupervised_loss, jnp.array(ROUNDS - 1, jnp.int32)
